# baseline (speedup 1.0000x reference)
import jax
import jax.numpy as jnp
from jax import lax
from jax.experimental import pallas as pl
from jax.experimental.pallas import tpu as pltpu
from jax.experimental.pallas import tpu_sc as plsc

_T = 256


def _gate_body(x_ref, wg_ref, s_ref, xb_ref):
    xv = x_ref[...]
    s_ref[...] = jnp.dot(xv, wg_ref[...], preferred_element_type=jnp.float32)
    xb_ref[...] = xv.astype(jnp.bfloat16)


def _gate(xf, Wg, bg):
    N, D = xf.shape
    E = Wg.shape[1]
    TG = 512
    wg_pad = jnp.zeros((D, 128), Wg.dtype).at[:, :E].set(Wg)
    scores, xb = pl.pallas_call(
        _gate_body,
        grid=(N // TG,),
        in_specs=[
            pl.BlockSpec((TG, D), lambda i: (i, 0)),
            pl.BlockSpec((D, 128), lambda i: (0, 0)),
        ],
        out_specs=[
            pl.BlockSpec((TG, 128), lambda i: (i, 0)),
            pl.BlockSpec((TG, D), lambda i: (i, 0)),
        ],
        out_shape=[
            jax.ShapeDtypeStruct((N, 128), jnp.float32),
            jax.ShapeDtypeStruct((N, D), jnp.bfloat16),
        ],
    )(xf, wg_pad)
    return scores[:, :E] + bg, xb


def _ffn_body(eid_ref, xs_ref, w1_ref, b1_ref, w2_ref, b2_ref, y_ref):
    xv = xs_ref[...]
    h = jnp.dot(xv, w1_ref[0].astype(jnp.bfloat16),
                preferred_element_type=jnp.float32)
    h = jnp.maximum(h + b1_ref[0, 0], 0.0)
    y = jnp.dot(h.astype(jnp.bfloat16), w2_ref[0].astype(jnp.bfloat16),
                preferred_element_type=jnp.float32)
    y_ref[...] = y + b2_ref[0, 0]


def _ffn(xs, tile_eid, W1, b1, W2, b2):
    NPAD, D = xs.shape
    E, _, H = W1.shape
    NT = NPAD // _T
    grid_spec = pltpu.PrefetchScalarGridSpec(
        num_scalar_prefetch=1,
        grid=(NT,),
        in_specs=[
            pl.BlockSpec((_T, D), lambda i, eid: (i, 0)),
            pl.BlockSpec((1, D, H), lambda i, eid: (eid[i], 0, 0)),
            pl.BlockSpec((1, 1, H), lambda i, eid: (eid[i], 0, 0)),
            pl.BlockSpec((1, H, D), lambda i, eid: (eid[i], 0, 0)),
            pl.BlockSpec((1, 1, D), lambda i, eid: (eid[i], 0, 0)),
        ],
        out_specs=pl.BlockSpec((_T, D), lambda i, eid: (i, 0)),
    )
    return pl.pallas_call(
        _ffn_body,
        grid_spec=grid_spec,
        out_shape=jax.ShapeDtypeStruct((NPAD, D), jnp.float32),
    )(tile_eid, xs, W1, b1.reshape(E, 1, H), W2, b2.reshape(E, 1, D))


_NW = 32


def _sc_mesh():
    return plsc.VectorSubcoreMesh(core_axis_name="c", subcore_axis_name="s")


def _sc_gather(table, idx):
    V, D = table.shape
    B = idx.shape[0]
    bpw = B // _NW

    def body(tab_hbm, idx_hbm, out_hbm, idx_v, rows_v, sem):
        wid = lax.axis_index("s") * 2 + lax.axis_index("c")
        base = wid * bpw
        pltpu.sync_copy(idx_hbm.at[pl.ds(base, bpw)], idx_v)
        pltpu.async_copy(tab_hbm.at[idx_v], rows_v, sem).wait()
        pltpu.sync_copy(rows_v, out_hbm.at[pl.ds(base, bpw)])

    return pl.kernel(
        body,
        out_type=jax.ShapeDtypeStruct((B, D), table.dtype),
        mesh=_sc_mesh(),
        scratch_types=[
            pltpu.VMEM((bpw,), jnp.int32),
            pltpu.VMEM((bpw, D), table.dtype),
            pltpu.SemaphoreType.DMA,
        ],
    )(table, idx)


def _sc_scatter(rows, idx, V):
    B, D = rows.shape
    bpw = B // _NW
    nch = 2
    ch = bpw // nch

    def body(rows_hbm, idx_hbm, out_hbm, idx_v, rows_v, sem):
        wid = lax.axis_index("s") * 2 + lax.axis_index("c")
        for c in range(nch):
            base = wid * bpw + c * ch
            pltpu.sync_copy(idx_hbm.at[pl.ds(base, ch)], idx_v)
            pltpu.sync_copy(rows_hbm.at[pl.ds(base, ch)], rows_v)
            pltpu.async_copy(rows_v, out_hbm.at[idx_v], sem).wait()

    return pl.kernel(
        body,
        out_type=jax.ShapeDtypeStruct((V, D), rows.dtype),
        mesh=_sc_mesh(),
        scratch_types=[
            pltpu.VMEM((ch,), jnp.int32),
            pltpu.VMEM((ch, D), rows.dtype),
            pltpu.SemaphoreType.DMA,
        ],
    )(rows, idx)


def kernel(x, Wg, bg, W1, b1, W2, b2):
    B, S, D = x.shape
    E = Wg.shape[1]
    N = B * S
    NT = N // _T + E
    NPAD = NT * _T

    xf = x.reshape(N, D)
    scores, xb = _gate(xf, Wg, bg)
    eid = jnp.argmax(scores, axis=1).astype(jnp.int32)

    perm_s = jnp.argsort(eid, stable=True).astype(jnp.int32)
    oh = eid[:, None] == jnp.arange(E, dtype=jnp.int32)[None, :]
    counts = jnp.sum(oh.astype(jnp.int32), axis=0)
    offs = jnp.concatenate([jnp.zeros((1,), jnp.int32),
                            jnp.cumsum(counts)[:-1].astype(jnp.int32)])
    tiles_per_e = (counts + _T - 1) // _T
    tile_bound = jnp.cumsum(tiles_per_e).astype(jnp.int32)
    t_idx = jnp.arange(NT, dtype=jnp.int32)
    tile_eid = jnp.sum((t_idx[:, None] >= tile_bound[None, :]).astype(jnp.int32),
                       axis=1)
    tile_eid = jnp.minimum(tile_eid, E - 1)
    pad_off = jnp.concatenate([jnp.zeros((1,), jnp.int32),
                               tile_bound[:-1]]) * _T
    j = jnp.arange(NPAD, dtype=jnp.int32)
    e_j = tile_eid[j // _T]
    src = j - (pad_off[e_j] - offs[e_j])
    valid = (j - pad_off[e_j]) < counts[e_j]
    perm_g = jnp.where(valid, perm_s[jnp.clip(src, 0, N - 1)], 0)
    perm_o = jnp.where(valid, perm_g, N)

    xb32 = lax.bitcast_convert_type(xb.reshape(N, D // 2, 2), jnp.int32)
    xs32 = _sc_gather(xb32, perm_g)
    xs = lax.bitcast_convert_type(xs32, jnp.bfloat16).reshape(NPAD, D)
    ys = _ffn(xs, tile_eid, W1, b1, W2, b2)
    out = _sc_scatter(ys, perm_o, N + 1)[:N]
    return (out.reshape(B, S, D), scores.reshape(B, S, E))


# speedup vs baseline: 1.3730x; 1.3730x over previous
import jax
import jax.numpy as jnp
from jax import lax
from jax.experimental import pallas as pl
from jax.experimental.pallas import tpu as pltpu
from jax.experimental.pallas import tpu_sc as plsc

_T = 256


def _gate_body(x_ref, wg_ref, s_ref, xb_ref):
    xv = x_ref[...]
    s_ref[...] = jnp.dot(xv, wg_ref[...], preferred_element_type=jnp.float32)
    xb_ref[...] = xv.astype(jnp.bfloat16)


def _gate(xf, Wg, bg):
    N, D = xf.shape
    E = Wg.shape[1]
    TG = 512
    wg_pad = jnp.zeros((D, 128), Wg.dtype).at[:, :E].set(Wg)
    scores, xb = pl.pallas_call(
        _gate_body,
        grid=(N // TG,),
        in_specs=[
            pl.BlockSpec((TG, D), lambda i: (i, 0)),
            pl.BlockSpec((D, 128), lambda i: (0, 0)),
        ],
        out_specs=[
            pl.BlockSpec((TG, 128), lambda i: (i, 0)),
            pl.BlockSpec((TG, D), lambda i: (i, 0)),
        ],
        out_shape=[
            jax.ShapeDtypeStruct((N, 128), jnp.float32),
            jax.ShapeDtypeStruct((N, D), jnp.bfloat16),
        ],
    )(xf, wg_pad)
    return scores[:, :E] + bg, xb


def _ffn_body(eid_ref, xs_ref, w1_ref, b1_ref, w2_ref, b2_ref, y_ref):
    xv = xs_ref[...]
    h = jnp.dot(xv, w1_ref[0].astype(jnp.bfloat16),
                preferred_element_type=jnp.float32)
    h = jnp.maximum(h + b1_ref[0, 0], 0.0)
    y = jnp.dot(h.astype(jnp.bfloat16), w2_ref[0].astype(jnp.bfloat16),
                preferred_element_type=jnp.float32)
    y_ref[...] = y + b2_ref[0, 0]


def _ffn(xs, tile_eid, W1, b1, W2, b2):
    NPAD, D = xs.shape
    E, _, H = W1.shape
    NT = NPAD // _T
    grid_spec = pltpu.PrefetchScalarGridSpec(
        num_scalar_prefetch=1,
        grid=(NT,),
        in_specs=[
            pl.BlockSpec((_T, D), lambda i, eid: (i, 0)),
            pl.BlockSpec((1, D, H), lambda i, eid: (eid[i], 0, 0)),
            pl.BlockSpec((1, 1, H), lambda i, eid: (eid[i], 0, 0)),
            pl.BlockSpec((1, H, D), lambda i, eid: (eid[i], 0, 0)),
            pl.BlockSpec((1, 1, D), lambda i, eid: (eid[i], 0, 0)),
        ],
        out_specs=pl.BlockSpec((_T, D), lambda i, eid: (i, 0)),
    )
    return pl.pallas_call(
        _ffn_body,
        grid_spec=grid_spec,
        out_shape=jax.ShapeDtypeStruct((NPAD, D), jnp.float32),
    )(tile_eid, xs, W1, b1.reshape(E, 1, H), W2, b2.reshape(E, 1, D))


_NW = 32


def _sc_mesh():
    return plsc.VectorSubcoreMesh(core_axis_name="c", subcore_axis_name="s")


def _sc_gather(table, idx):
    V, D = table.shape
    B = idx.shape[0]
    bpw = B // _NW

    nch = 4
    ch = bpw // nch

    def body(tab_hbm, idx_hbm, out_hbm, idx_v, b0, b1, gsem, osem):
        wid = lax.axis_index("s") * 2 + lax.axis_index("c")
        base = wid * bpw
        pltpu.sync_copy(idx_hbm.at[pl.ds(base, bpw)], idx_v)
        bufs = (b0, b1)
        g = [None] * nch
        o = [None] * nch
        g[0] = pltpu.async_copy(tab_hbm.at[idx_v.at[pl.ds(0, ch)]], bufs[0],
                                gsem.at[0])
        for c in range(nch):
            b = c % 2
            g[c].wait()
            if c >= 1:
                o[c - 1].wait()
            o[c] = pltpu.async_copy(bufs[b], out_hbm.at[pl.ds(base + c * ch, ch)],
                                    osem.at[b])
            if c + 1 < nch:
                g[c + 1] = pltpu.async_copy(
                    tab_hbm.at[idx_v.at[pl.ds((c + 1) * ch, ch)]],
                    bufs[1 - b], gsem.at[1 - b])
        o[nch - 1].wait()

    return pl.kernel(
        body,
        out_type=jax.ShapeDtypeStruct((B, D), table.dtype),
        mesh=_sc_mesh(),
        scratch_types=[
            pltpu.VMEM((bpw,), jnp.int32),
            pltpu.VMEM((ch, D), table.dtype),
            pltpu.VMEM((ch, D), table.dtype),
            pltpu.SemaphoreType.DMA((2,)),
            pltpu.SemaphoreType.DMA((2,)),
        ],
    )(table, idx)


def _sc_scatter(rows, idx, V):
    B, D = rows.shape
    bpw = B // _NW
    nch = 4
    ch = bpw // nch
    idx2 = idx.reshape(_NW * nch, ch)

    def body(rows_hbm, idx_hbm, out_hbm, idx_v, b0, b1, rsem, wsem):
        wid = lax.axis_index("s") * 2 + lax.axis_index("c")
        base = wid * bpw
        pltpu.sync_copy(idx_hbm.at[pl.ds(wid * nch, nch)], idx_v)
        bufs = (b0, b1)
        r = [None] * nch
        w = [None] * nch
        r[0] = pltpu.async_copy(rows_hbm.at[pl.ds(base, ch)], bufs[0],
                                rsem.at[0])
        for c in range(nch):
            b = c % 2
            r[c].wait()
            if c >= 1:
                w[c - 1].wait()
            w[c] = pltpu.async_copy(bufs[b], out_hbm.at[idx_v.at[c]],
                                    wsem.at[b])
            if c + 1 < nch:
                r[c + 1] = pltpu.async_copy(
                    rows_hbm.at[pl.ds(base + (c + 1) * ch, ch)],
                    bufs[1 - b], rsem.at[1 - b])
        w[nch - 1].wait()

    return pl.kernel(
        body,
        out_type=jax.ShapeDtypeStruct((V, D), rows.dtype),
        mesh=_sc_mesh(),
        scratch_types=[
            pltpu.VMEM((nch, ch), jnp.int32),
            pltpu.VMEM((ch, D), rows.dtype),
            pltpu.VMEM((ch, D), rows.dtype),
            pltpu.SemaphoreType.DMA((2,)),
            pltpu.SemaphoreType.DMA((2,)),
        ],
    )(rows, idx2)


def kernel(x, Wg, bg, W1, b1, W2, b2):
    B, S, D = x.shape
    E = Wg.shape[1]
    N = B * S
    NT = N // _T + E
    NPAD = NT * _T

    xf = x.reshape(N, D)
    scores, xb = _gate(xf, Wg, bg)
    eid = jnp.argmax(scores, axis=1).astype(jnp.int32)

    perm_s = jnp.argsort(eid, stable=True).astype(jnp.int32)
    oh = eid[:, None] == jnp.arange(E, dtype=jnp.int32)[None, :]
    counts = jnp.sum(oh.astype(jnp.int32), axis=0)
    offs = jnp.concatenate([jnp.zeros((1,), jnp.int32),
                            jnp.cumsum(counts)[:-1].astype(jnp.int32)])
    tiles_per_e = (counts + _T - 1) // _T
    tile_bound = jnp.cumsum(tiles_per_e).astype(jnp.int32)
    t_idx = jnp.arange(NT, dtype=jnp.int32)
    tile_eid = jnp.sum((t_idx[:, None] >= tile_bound[None, :]).astype(jnp.int32),
                       axis=1)
    tile_eid = jnp.minimum(tile_eid, E - 1)
    pad_off = jnp.concatenate([jnp.zeros((1,), jnp.int32),
                               tile_bound[:-1]]) * _T
    j = jnp.arange(NPAD, dtype=jnp.int32)
    e_j = tile_eid[j // _T]
    src = j - (pad_off[e_j] - offs[e_j])
    valid = (j - pad_off[e_j]) < counts[e_j]
    perm_g = jnp.where(valid, perm_s[jnp.clip(src, 0, N - 1)], j & (N - 1))
    perm_o = jnp.where(valid, perm_g, N + (j & (NPAD - N - 1)))

    xb32 = lax.bitcast_convert_type(xb.reshape(N, D // 2, 2), jnp.int32)
    xs32 = _sc_gather(xb32, perm_g)
    xs = lax.bitcast_convert_type(xs32, jnp.bfloat16).reshape(NPAD, D)
    ys = _ffn(xs, tile_eid, W1, b1, W2, b2)
    out = _sc_scatter(ys, perm_o, NPAD)[:N]
    return (out.reshape(B, S, D), scores.reshape(B, S, E))


# speedup vs baseline: 4.0162x; 2.9252x over previous
import jax
import jax.numpy as jnp
from jax import lax
from jax.experimental import pallas as pl
from jax.experimental.pallas import tpu as pltpu
from jax.experimental.pallas import tpu_sc as plsc

_T = 256
_NW = 32


def _gate_body(x_ref, wg_ref, s_ref):
    s_ref[...] = jnp.dot(x_ref[...], wg_ref[...],
                         preferred_element_type=jnp.float32)


def _gate(xf, Wg, bg):
    N, D = xf.shape
    E = Wg.shape[1]
    TG = 512
    wg_pad = jnp.zeros((D, 128), Wg.dtype).at[:, :E].set(Wg)
    scores = pl.pallas_call(
        _gate_body,
        grid=(N // TG,),
        in_specs=[
            pl.BlockSpec((TG, D), lambda i: (i, 0)),
            pl.BlockSpec((D, 128), lambda i: (0, 0)),
        ],
        out_specs=pl.BlockSpec((TG, 128), lambda i: (i, 0)),
        out_shape=jax.ShapeDtypeStruct((N, 128), jnp.float32),
    )(xf, wg_pad)
    return scores[:, :E] + bg


def _ffn_body(eid_ref, xs_ref, w1_ref, b1_ref, w2_ref, b2_ref, y_ref):
    xv = xs_ref[...].astype(jnp.bfloat16)
    h = jnp.dot(xv, w1_ref[0].astype(jnp.bfloat16),
                preferred_element_type=jnp.float32)
    h = jnp.maximum(h + b1_ref[0, 0], 0.0)
    y = jnp.dot(h.astype(jnp.bfloat16), w2_ref[0].astype(jnp.bfloat16),
                preferred_element_type=jnp.float32)
    y_ref[...] = y + b2_ref[0, 0]


def _ffn(xs, tile_eid, W1, b1, W2, b2):
    NPAD, D = xs.shape
    E, _, H = W1.shape
    NT = NPAD // _T
    grid_spec = pltpu.PrefetchScalarGridSpec(
        num_scalar_prefetch=1,
        grid=(NT,),
        in_specs=[
            pl.BlockSpec((_T, D), lambda i, eid: (i, 0)),
            pl.BlockSpec((1, D, H), lambda i, eid: (eid[i], 0, 0)),
            pl.BlockSpec((1, 1, H), lambda i, eid: (eid[i], 0, 0)),
            pl.BlockSpec((1, H, D), lambda i, eid: (eid[i], 0, 0)),
            pl.BlockSpec((1, 1, D), lambda i, eid: (eid[i], 0, 0)),
        ],
        out_specs=pl.BlockSpec((_T, D), lambda i, eid: (i, 0)),
    )
    return pl.pallas_call(
        _ffn_body,
        grid_spec=grid_spec,
        out_shape=jax.ShapeDtypeStruct((NPAD, D), jnp.float32),
    )(tile_eid, xs, W1, b1.reshape(E, 1, H), W2, b2.reshape(E, 1, D))


def _sc_mesh():
    return plsc.VectorSubcoreMesh(core_axis_name="c", subcore_axis_name="s")


def _sc_gather(table, idx):
    V, D = table.shape
    B = idx.shape[0]
    bpw = B // _NW
    nch = 4
    ch = bpw // nch

    def body(tab_hbm, idx_hbm, out_hbm, idx_v, b0, b1, gsem, osem):
        wid = lax.axis_index("s") * 2 + lax.axis_index("c")
        base = wid * bpw
        pltpu.sync_copy(idx_hbm.at[pl.ds(base, bpw)], idx_v)
        bufs = (b0, b1)
        g = [None] * nch
        o = [None] * nch
        g[0] = pltpu.async_copy(tab_hbm.at[idx_v.at[pl.ds(0, ch)]], bufs[0],
                                gsem.at[0])
        for c in range(nch):
            b = c % 2
            g[c].wait()
            if c >= 1:
                o[c - 1].wait()
            o[c] = pltpu.async_copy(bufs[b], out_hbm.at[pl.ds(base + c * ch, ch)],
                                    osem.at[b])
            if c + 1 < nch:
                g[c + 1] = pltpu.async_copy(
                    tab_hbm.at[idx_v.at[pl.ds((c + 1) * ch, ch)]],
                    bufs[1 - b], gsem.at[1 - b])
        o[nch - 1].wait()

    return pl.kernel(
        body,
        out_type=jax.ShapeDtypeStruct((B, D), table.dtype),
        mesh=_sc_mesh(),
        scratch_types=[
            pltpu.VMEM((bpw,), jnp.int32),
            pltpu.VMEM((ch, D), table.dtype),
            pltpu.VMEM((ch, D), table.dtype),
            pltpu.SemaphoreType.DMA((2,)),
            pltpu.SemaphoreType.DMA((2,)),
        ],
    )(table, idx)


def _sc_scatter(rows, idx, V):
    B, D = rows.shape
    bpw = B // _NW
    nch = 4
    ch = bpw // nch
    idx2 = idx.reshape(_NW * nch, ch)

    def body(rows_hbm, idx_hbm, out_hbm, idx_v, b0, b1, rsem, wsem):
        wid = lax.axis_index("s") * 2 + lax.axis_index("c")
        base = wid * bpw
        pltpu.sync_copy(idx_hbm.at[pl.ds(wid * nch, nch)], idx_v)
        bufs = (b0, b1)
        r = [None] * nch
        w = [None] * nch
        r[0] = pltpu.async_copy(rows_hbm.at[pl.ds(base, ch)], bufs[0],
                                rsem.at[0])
        for c in range(nch):
            b = c % 2
            r[c].wait()
            if c >= 1:
                w[c - 1].wait()
            w[c] = pltpu.async_copy(bufs[b], out_hbm.at[idx_v.at[c]],
                                    wsem.at[b])
            if c + 1 < nch:
                r[c + 1] = pltpu.async_copy(
                    rows_hbm.at[pl.ds(base + (c + 1) * ch, ch)],
                    bufs[1 - b], rsem.at[1 - b])
        w[nch - 1].wait()

    return pl.kernel(
        body,
        out_type=jax.ShapeDtypeStruct((V, D), rows.dtype),
        mesh=_sc_mesh(),
        scratch_types=[
            pltpu.VMEM((nch, ch), jnp.int32),
            pltpu.VMEM((ch, D), rows.dtype),
            pltpu.VMEM((ch, D), rows.dtype),
            pltpu.SemaphoreType.DMA((2,)),
            pltpu.SemaphoreType.DMA((2,)),
        ],
    )(rows, idx2)


def kernel(x, Wg, bg, W1, b1, W2, b2):
    B, S, D = x.shape
    E = Wg.shape[1]
    N = B * S
    NT = N // _T + E
    NPAD = NT * _T

    xf = x.reshape(N, D)
    scores = _gate(xf, Wg, bg)
    eid = jnp.argmax(scores, axis=1).astype(jnp.int32)

    oh = (eid[:, None] == jnp.arange(E, dtype=jnp.int32)[None, :]).astype(
        jnp.int32)
    cc = jnp.cumsum(oh, axis=0)
    counts = cc[-1]
    rank = jnp.take_along_axis(cc, eid[:, None], axis=1)[:, 0] - 1
    tiles_per_e = (counts + _T - 1) // _T
    tile_bound = jnp.cumsum(tiles_per_e).astype(jnp.int32)
    pad_off = jnp.concatenate([jnp.zeros((1,), jnp.int32),
                               tile_bound[:-1]]) * _T
    pos = pad_off[eid] + rank
    t_idx = jnp.arange(NT, dtype=jnp.int32)
    tile_eid = jnp.sum((t_idx[:, None] >= tile_bound[None, :]).astype(
        jnp.int32), axis=1)
    tile_eid = jnp.minimum(tile_eid, E - 1)

    xs = _sc_scatter(xf, pos, NPAD)
    ys = _ffn(xs, tile_eid, W1, b1, W2, b2)
    out = _sc_gather(ys, pos)
    return (out.reshape(B, S, D), scores.reshape(B, S, E))
